# baseline (speedup 1.0000x reference)
import functools

import jax
import jax.numpy as jnp
from jax import lax
from jax.experimental import pallas as pl
from jax.experimental.pallas import tpu as pltpu
from jax.experimental.pallas import tpu_sc as plsc

B = 16384
D = 64
NC = 2
NS = 16
NW = NC * NS
BPW = B // NW
CHUNK = 128
NCH = BPW // CHUNK
LANES = 16


@functools.partial(
    pl.kernel,
    mesh=plsc.VectorSubcoreMesh(core_axis_name="c", subcore_axis_name="s"),
    out_type=jax.ShapeDtypeStruct((NW, LANES), jnp.float32),
    compiler_params=pltpu.CompilerParams(use_tc_tiling_on_sc=False),
    scratch_types=[
        pltpu.VMEM((NCH, CHUNK), jnp.int32),
        pltpu.VMEM((BPW, D), jnp.float32),
        pltpu.VMEM((BPW, D), jnp.float32),
        pltpu.VMEM((LANES,), jnp.float32),
        pltpu.SemaphoreType.DMA,
        pltpu.SemaphoreType.DMA,
    ],
)
def _centerloss_partials(feat_hbm, label_hbm, centers_hbm, out_hbm,
                         idx_v, cent_v, feat_v, res_v, gsem, fsem):
    wid = lax.axis_index("s") * NC + lax.axis_index("c")
    base = wid * BPW

    pltpu.sync_copy(label_hbm.at[wid], idx_v)
    fcp = pltpu.async_copy(feat_hbm.at[pl.ds(base, BPW)], feat_v, fsem)
    gcps = []
    for j in range(NCH):
        gcps.append(pltpu.async_copy(
            centers_hbm.at[idx_v.at[j]],
            cent_v.at[pl.ds(j * CHUNK, CHUNK)],
            gsem,
        ))
    fcp.wait()
    for cp in gcps:
        cp.wait()

    def body(r, acc):
        for c in range(D // LANES):
            f = feat_v[r, pl.ds(c * LANES, LANES)]
            g = cent_v[r, pl.ds(c * LANES, LANES)]
            dlt = f - g
            acc = acc + dlt * dlt
        return acc

    acc = lax.fori_loop(0, BPW, body, jnp.zeros((LANES,), jnp.float32))
    res_v[...] = acc
    pltpu.sync_copy(res_v, out_hbm.at[wid])


def kernel(feature, label, centers, batch_size):
    label3 = label.astype(jnp.int32).reshape(NW, NCH, CHUNK)
    partials = _centerloss_partials(feature, label3, centers)
    return jnp.sum(partials) / 2.0 / batch_size


# speedup vs baseline: 1.2876x; 1.2876x over previous
import functools

import jax
import jax.numpy as jnp
from jax import lax
from jax.experimental import pallas as pl
from jax.experimental.pallas import tpu as pltpu
from jax.experimental.pallas import tpu_sc as plsc

B = 16384
D = 64
NC = 2
NS = 16
NW = NC * NS
BPW = B // NW
LANES = 16
K = 64
NG = BPW // K


@functools.partial(
    pl.kernel,
    mesh=plsc.VectorSubcoreMesh(core_axis_name="c", subcore_axis_name="s"),
    out_type=jax.ShapeDtypeStruct((NW, LANES), jnp.float32),
    scratch_types=[
        pltpu.VMEM((4, 128), jnp.int32),
        pltpu.VMEM((K, D), jnp.float32),
        pltpu.VMEM((K, D), jnp.float32),
        pltpu.VMEM((K, D), jnp.float32),
        pltpu.VMEM((K, D), jnp.float32),
        pltpu.VMEM((1, LANES), jnp.float32),
        pltpu.SemaphoreType.DMA,
        pltpu.SemaphoreType.DMA,
        pltpu.SemaphoreType.DMA,
        pltpu.SemaphoreType.DMA,
    ],
)
def _centerloss_partials(feat_hbm, label_hbm, centers_hbm, out_hbm,
                         idx_v, cb0, cb1, fb0, fb1, res_v,
                         gs0, gs1, fs0, fs1):
    wid = lax.axis_index("s") * NC + lax.axis_index("c")
    base = wid * BPW
    cbufs, fbufs = (cb0, cb1), (fb0, fb1)
    gsems, fsems = (gs0, gs1), (fs0, fs1)

    pltpu.sync_copy(label_hbm.at[wid], idx_v)

    def fire(g):
        p = g % 2
        descs = [pltpu.async_copy(
            feat_hbm.at[pl.ds(base + g * K, K)], fbufs[p], fsems[p])]
        for q in range(K // LANES):
            vec = idx_v[(g * K + q * LANES) // 128,
                        pl.ds((g * K + q * LANES) % 128, LANES)]
            for j in range(LANES):
                lbl = vec[j]
                descs.append(pltpu.async_copy(
                    centers_hbm.at[pl.ds(lbl, 1)],
                    cbufs[p].at[pl.ds(q * LANES + j, 1)], gsems[p]))
        return descs

    def compute(g, acc):
        p = g % 2
        fb, cb = fbufs[p], cbufs[p]

        def body(r, a):
            for c in range(D // LANES):
                f = fb[r, pl.ds(c * LANES, LANES)]
                gsl = cb[r, pl.ds(c * LANES, LANES)]
                dlt = f - gsl
                a = a + dlt * dlt
            return a

        return lax.fori_loop(0, K, body, acc)

    acc = jnp.zeros((LANES,), jnp.float32)
    descs = fire(0)
    for g in range(NG):
        nxt = fire(g + 1) if g + 1 < NG else []
        for d in descs:
            d.wait()
        acc = compute(g, acc)
        descs = nxt

    res_v[0, :] = acc
    pltpu.sync_copy(res_v, out_hbm.at[pl.ds(wid, 1)])


def kernel(feature, label, centers, batch_size):
    label3 = label.astype(jnp.int32).reshape(NW, 4, 128)
    partials = _centerloss_partials(feature, label3, centers)
    return jnp.sum(partials) / 2.0 / batch_size


# speedup vs baseline: 1.3604x; 1.0565x over previous
import functools

import jax
import jax.numpy as jnp
from jax import lax
from jax.experimental import pallas as pl
from jax.experimental.pallas import tpu as pltpu
from jax.experimental.pallas import tpu_sc as plsc

B = 16384
D = 64
NC = 2
NS = 16
NW = NC * NS
BPW = B // NW
LANES = 16
K = 64
NG = BPW // K


@functools.partial(
    pl.kernel,
    mesh=plsc.VectorSubcoreMesh(core_axis_name="c", subcore_axis_name="s"),
    out_type=jax.ShapeDtypeStruct((NW, LANES), jnp.float32),
    scratch_types=[
        pltpu.VMEM((4, 128), jnp.int32),
        pltpu.VMEM((K, D), jnp.float32),
        pltpu.VMEM((K, D), jnp.float32),
        pltpu.VMEM((K, D), jnp.float32),
        pltpu.VMEM((K, D), jnp.float32),
        pltpu.VMEM((1, LANES), jnp.float32),
        pltpu.SemaphoreType.DMA,
        pltpu.SemaphoreType.DMA,
        pltpu.SemaphoreType.DMA,
        pltpu.SemaphoreType.DMA,
    ],
)
def _centerloss_partials(feat_hbm, label_hbm, centers_hbm, out_hbm,
                         idx_v, cb0, cb1, fb0, fb1, res_v,
                         gs0, gs1, fs0, fs1):
    wid = lax.axis_index("s") * NC + lax.axis_index("c")
    base = wid * BPW
    cbufs, fbufs = (cb0, cb1), (fb0, fb1)
    gsems, fsems = (gs0, gs1), (fs0, fs1)

    pltpu.sync_copy(label_hbm.at[wid], idx_v)

    def fire(g, p):
        pltpu.async_copy(feat_hbm.at[pl.ds(base + g * K, K)],
                         fbufs[p], fsems[p])
        for q in range(K // LANES):
            i0 = g * K + q * LANES
            vec = idx_v[i0 // 128, pl.ds(i0 % 128, LANES)]
            for j in range(LANES):
                lbl = vec[j]
                pltpu.async_copy(
                    centers_hbm.at[pl.ds(lbl, 1)],
                    cbufs[p].at[pl.ds(q * LANES + j, 1)], gsems[p])

    def drain(p):
        pltpu.make_async_copy(feat_hbm.at[pl.ds(0, K)],
                              fbufs[p], fsems[p]).wait()
        for _ in range(K):
            pltpu.make_async_copy(centers_hbm.at[pl.ds(0, 1)],
                                  cbufs[p].at[pl.ds(0, 1)], gsems[p]).wait()

    def compute(p, acc):
        fb, cb = fbufs[p], cbufs[p]

        def body(r, a):
            for c in range(D // LANES):
                f = fb[r, pl.ds(c * LANES, LANES)]
                gsl = cb[r, pl.ds(c * LANES, LANES)]
                dlt = f - gsl
                a = a + dlt * dlt
            return a

        return lax.fori_loop(0, K, body, acc)

    fire(0, 0)

    def gbody(gg, acc):
        g0 = 2 * gg
        fire(g0 + 1, 1)
        drain(0)
        acc = compute(0, acc)

        @pl.when(g0 + 2 < NG)
        def _():
            fire(g0 + 2, 0)

        drain(1)
        acc = compute(1, acc)
        return acc

    acc = lax.fori_loop(0, NG // 2, gbody, jnp.zeros((LANES,), jnp.float32))

    res_v[0, :] = acc
    pltpu.sync_copy(res_v, out_hbm.at[pl.ds(wid, 1)])


def kernel(feature, label, centers, batch_size):
    label3 = label.astype(jnp.int32).reshape(NW, 4, 128)
    partials = _centerloss_partials(feature, label3, centers)
    return jnp.sum(partials) / 2.0 / batch_size
